# baseline (speedup 1.0000x reference)
import functools

import jax
import jax.numpy as jnp
from jax import lax
from jax.experimental import pallas as pl
from jax.experimental.pallas import tpu as pltpu
from jax.experimental.pallas import tpu_sc as plsc

B = 4096
D = 32
LANES = 16

_info = plsc.get_sparse_core_info()
_NC, _NS = _info.num_cores, _info.num_subcores
_NW = _NC * _NS
_NB = 16

_mesh = plsc.VectorSubcoreMesh(core_axis_name="c", subcore_axis_name="s")


def _make_gather(n, transposed_out):
    bpw = n // _NW
    ng = bpw // _NB
    out_shape = (D, n) if transposed_out else (n, D)
    loc_shape = (D, bpw) if transposed_out else (bpw, D)

    @functools.partial(
        pl.kernel,
        mesh=_mesh,
        compiler_params=pltpu.CompilerParams(needs_layout_passes=False),
        out_type=jax.ShapeDtypeStruct(out_shape, jnp.float32),
        scratch_types=[
            pltpu.VMEM((bpw + LANES,), jnp.int32),
            pltpu.VMEM((_NB, D, 128), jnp.float32),
            pltpu.VMEM(loc_shape, jnp.float32),
            [pltpu.SemaphoreType.DMA] * _NB,
        ],
    )
    def gather(idx_hbm, tab_hbm, out_hbm, idx_v, buf, loc, sems):
        wid = lax.axis_index("s") * _NC + lax.axis_index("c")
        base = wid * bpw
        pltpu.sync_copy(idx_hbm.at[pl.ds(base, bpw)], idx_v.at[pl.ds(0, bpw)])

        def _fetch(j, b):
            c = pl.multiple_of(jnp.bitwise_and(j, -128), 128)
            pltpu.async_copy(tab_hbm.at[:, pl.ds(c, 128)], buf.at[b], sems[b])

        def _drain(b):
            pltpu.make_async_copy(
                tab_hbm.at[:, pl.ds(0, 128)], buf.at[b], sems[b]).wait()

        iota_lo = lax.iota(jnp.int32, LANES)
        iota_hi = iota_lo + LANES

        def _select(b, lane, k):
            lane_v = jnp.full((LANES,), lane, jnp.int32)
            lo = plsc.load_gather(buf.at[b], [iota_lo, lane_v])
            hi = plsc.load_gather(buf.at[b], [iota_hi, lane_v])
            if transposed_out:
                k_v = jnp.full((LANES,), k, jnp.int32)
                plsc.store_scatter(loc, [iota_lo, k_v], lo)
                plsc.store_scatter(loc, [iota_hi, k_v], hi)
            else:
                loc[k, pl.ds(0, LANES)] = lo
                loc[k, pl.ds(LANES, LANES)] = hi

        vec0 = idx_v[pl.ds(0, LANES)]
        for b in range(_NB):
            _fetch(vec0[b], b)

        def group(g, carry):
            cur = idx_v[pl.ds(g * _NB, LANES)]
            nxt = idx_v[pl.ds((g + 1) * _NB, LANES)]
            for b in range(_NB):
                k = g * _NB + b
                _drain(b)
                _select(b, jnp.bitwise_and(cur[b], 127), k)

                @pl.when(k + _NB < bpw)
                def _():
                    _fetch(nxt[b], b)

            return carry

        lax.fori_loop(0, ng, group, 0)
        if transposed_out:
            pltpu.sync_copy(loc, out_hbm.at[:, pl.ds(base, bpw)])
        else:
            pltpu.sync_copy(loc, out_hbm.at[pl.ds(base, bpw), :])

    return gather


_gather_t_full = _make_gather(B, True)
_gather_r_half = _make_gather(B // 2, False)


def _mm_body(a_ref, b_ref, o_ref):
    o_ref[...] = lax.dot_general(
        a_ref[...], b_ref[...],
        dimension_numbers=(((1,), (0,)), ((), ())),
        preferred_element_type=jnp.float32,
    )


_BM = 256
_HBLK = (B // 2) // _BM


def _mm_half1(u_ebd, t_ebd_t):
    return pl.pallas_call(
        _mm_body,
        grid=(_HBLK,),
        in_specs=[
            pl.BlockSpec((_BM, D), lambda i: (i, 0)),
            pl.BlockSpec((D, B), lambda i: (0, 0)),
        ],
        out_specs=pl.BlockSpec((_BM, B), lambda i: (i, 0)),
        out_shape=jax.ShapeDtypeStruct((B, B), jnp.float32),
    )(u_ebd, t_ebd_t)


def _mm_body2(a_ref, b_ref, prev_ref, o_ref):
    del prev_ref
    _mm_body(a_ref, b_ref, o_ref)


def _mm_half2(u_ebd, t_ebd_t, prev):
    return pl.pallas_call(
        _mm_body2,
        grid=(_HBLK,),
        in_specs=[
            pl.BlockSpec((_BM, D), lambda i: (i, 0)),
            pl.BlockSpec((D, B), lambda i: (0, 0)),
            pl.BlockSpec(memory_space=pl.ANY),
        ],
        out_specs=pl.BlockSpec((_BM, B), lambda i: (i + _HBLK, 0)),
        out_shape=jax.ShapeDtypeStruct((B, B), jnp.float32),
        input_output_aliases={2: 0},
    )(u_ebd, t_ebd_t, prev)


def kernel(user_indices, item_seq_indices, target_item_indices, target_domain, U, V):
    uidx = user_indices.astype(jnp.int32)
    tidx = target_item_indices.reshape(B).astype(jnp.int32)
    ut = U.T
    vt = V.T
    t_ebd_t = _gather_t_full(tidx, vt)
    u1 = _gather_r_half(uidx[: B // 2], ut)
    u2 = _gather_r_half(uidx[B // 2:], ut)
    s1 = _mm_half1(u1, t_ebd_t)
    return _mm_half2(u2, t_ebd_t, s1)


# speedup vs baseline: 1.0635x; 1.0635x over previous
import functools

import jax
import jax.numpy as jnp
from jax import lax
from jax.experimental import pallas as pl
from jax.experimental.pallas import tpu as pltpu
from jax.experimental.pallas import tpu_sc as plsc

B = 4096
D = 32
LANES = 16

_info = plsc.get_sparse_core_info()
_NC, _NS = _info.num_cores, _info.num_subcores
_NW = _NC * _NS
_BPW = B // _NW
_NB = 8
_NG = _BPW // _NB

_mesh = plsc.VectorSubcoreMesh(core_axis_name="c", subcore_axis_name="s")


@functools.partial(
    pl.kernel,
    mesh=_mesh,
    compiler_params=pltpu.CompilerParams(needs_layout_passes=False),
    out_type=[
        jax.ShapeDtypeStruct((B, D), jnp.float32),
        jax.ShapeDtypeStruct((D, B), jnp.float32),
    ],
    scratch_types=[
        pltpu.VMEM((_BPW + LANES,), jnp.int32),
        pltpu.VMEM((_BPW + LANES,), jnp.int32),
        pltpu.VMEM((_NB, D, 128), jnp.float32),
        pltpu.VMEM((_NB, D, 128), jnp.float32),
        pltpu.VMEM((_BPW, D), jnp.float32),
        pltpu.VMEM((D, _BPW), jnp.float32),
        [pltpu.SemaphoreType.DMA] * _NB,
        [pltpu.SemaphoreType.DMA] * _NB,
    ],
)
def _sc_gather(uidx_hbm, tidx_hbm, ut_hbm, vt_hbm, uout_hbm, tout_hbm,
               uidx_v, tidx_v, ubuf, tbuf, uoutR, toutT, usems, tsems):
    wid = lax.axis_index("s") * _NC + lax.axis_index("c")
    base = wid * _BPW
    pltpu.sync_copy(uidx_hbm.at[pl.ds(base, _BPW)], uidx_v.at[pl.ds(0, _BPW)])
    pltpu.sync_copy(tidx_hbm.at[pl.ds(base, _BPW)], tidx_v.at[pl.ds(0, _BPW)])

    def _fetch(ju, jt, b):
        cu = pl.multiple_of(jnp.bitwise_and(ju, -128), 128)
        ct = pl.multiple_of(jnp.bitwise_and(jt, -128), 128)
        pltpu.async_copy(ut_hbm.at[:, pl.ds(cu, 128)], ubuf.at[b], usems[b])
        pltpu.async_copy(vt_hbm.at[:, pl.ds(ct, 128)], tbuf.at[b], tsems[b])

    def _drain(b):
        pltpu.make_async_copy(ut_hbm.at[:, pl.ds(0, 128)], ubuf.at[b], usems[b]).wait()
        pltpu.make_async_copy(vt_hbm.at[:, pl.ds(0, 128)], tbuf.at[b], tsems[b]).wait()

    iota_lo = lax.iota(jnp.int32, LANES)
    iota_hi = iota_lo + LANES

    def _select_row(b, lane, k):
        lane_v = jnp.full((LANES,), lane, jnp.int32)
        lo = plsc.load_gather(ubuf.at[b], [iota_lo, lane_v])
        hi = plsc.load_gather(ubuf.at[b], [iota_hi, lane_v])
        uoutR[k, pl.ds(0, LANES)] = lo
        uoutR[k, pl.ds(LANES, LANES)] = hi

    def _select_col(b, lane, k):
        lane_v = jnp.full((LANES,), lane, jnp.int32)
        k_v = jnp.full((LANES,), k, jnp.int32)
        lo = plsc.load_gather(tbuf.at[b], [iota_lo, lane_v])
        hi = plsc.load_gather(tbuf.at[b], [iota_hi, lane_v])
        plsc.store_scatter(toutT, [iota_lo, k_v], lo)
        plsc.store_scatter(toutT, [iota_hi, k_v], hi)

    uvec0 = uidx_v[pl.ds(0, LANES)]
    tvec0 = tidx_v[pl.ds(0, LANES)]
    for b in range(_NB):
        _fetch(uvec0[b], tvec0[b], b)

    def group(g, carry):
        uvec = uidx_v[pl.ds(g * _NB, LANES)]
        tvec = tidx_v[pl.ds(g * _NB, LANES)]
        for b in range(_NB):
            k = g * _NB + b
            _drain(b)
            _select_row(b, jnp.bitwise_and(uvec[b], 127), k)
            _select_col(b, jnp.bitwise_and(tvec[b], 127), k)

            @pl.when(k + _NB < _BPW)
            def _():
                _fetch(uvec[b + _NB], tvec[b + _NB], b)

        return carry

    lax.fori_loop(0, _NG, group, 0)
    pltpu.sync_copy(uoutR, uout_hbm.at[pl.ds(base, _BPW), :])
    pltpu.sync_copy(toutT, tout_hbm.at[:, pl.ds(base, _BPW)])


def _mm_body(a_ref, b_ref, o_ref):
    o_ref[...] = lax.dot_general(
        a_ref[...], b_ref[...],
        dimension_numbers=(((1,), (0,)), ((), ())),
        preferred_element_type=jnp.float32,
    )


_BM = 256


def _tc_matmul(u_ebd, t_ebd_t):
    return pl.pallas_call(
        _mm_body,
        grid=(B // _BM,),
        in_specs=[
            pl.BlockSpec((_BM, D), lambda i: (i, 0)),
            pl.BlockSpec((D, B), lambda i: (0, 0)),
        ],
        out_specs=pl.BlockSpec((_BM, B), lambda i: (i, 0)),
        out_shape=jax.ShapeDtypeStruct((B, B), jnp.float32),
    )(u_ebd, t_ebd_t)


def kernel(user_indices, item_seq_indices, target_item_indices, target_domain, U, V):
    uidx = user_indices.astype(jnp.int32)
    tidx = target_item_indices.reshape(B).astype(jnp.int32)
    u_ebd, t_ebd_t = _sc_gather(uidx, tidx, U.T, V.T)
    return _tc_matmul(u_ebd, t_ebd_t)


# speedup vs baseline: 1.0871x; 1.0222x over previous
import functools

import jax
import jax.numpy as jnp
from jax import lax
from jax.experimental import pallas as pl
from jax.experimental.pallas import tpu as pltpu
from jax.experimental.pallas import tpu_sc as plsc

B = 4096
D = 32
LANES = 16

_info = plsc.get_sparse_core_info()
_NC, _NS = _info.num_cores, _info.num_subcores
_NW = _NC * _NS
_BPW = B // _NW
_NB = 8
_NG = _BPW // _NB

_mesh = plsc.VectorSubcoreMesh(core_axis_name="c", subcore_axis_name="s")


@functools.partial(
    pl.kernel,
    mesh=_mesh,
    compiler_params=pltpu.CompilerParams(needs_layout_passes=False),
    out_type=[
        jax.ShapeDtypeStruct((B, D), jnp.float32),
        jax.ShapeDtypeStruct((D, B), jnp.float32),
    ],
    scratch_types=[
        pltpu.VMEM((_BPW + LANES,), jnp.int32),
        pltpu.VMEM((_BPW + LANES,), jnp.int32),
        pltpu.VMEM((_NB, D, 128), jnp.float32),
        pltpu.VMEM((_NB, D, 128), jnp.float32),
        pltpu.VMEM((_BPW, D), jnp.float32),
        pltpu.VMEM((D, _BPW), jnp.float32),
        [pltpu.SemaphoreType.DMA] * _NB,
        [pltpu.SemaphoreType.DMA] * _NB,
    ],
)
def _sc_gather(uidx_hbm, tidx_hbm, ut_hbm, vt_hbm, uout_hbm, tout_hbm,
               uidx_v, tidx_v, ubuf, tbuf, uoutR, toutT, usems, tsems):
    wid = lax.axis_index("s") * _NC + lax.axis_index("c")
    base = wid * _BPW
    pltpu.sync_copy(uidx_hbm.at[pl.ds(base, _BPW)], uidx_v.at[pl.ds(0, _BPW)])
    pltpu.sync_copy(tidx_hbm.at[pl.ds(base, _BPW)], tidx_v.at[pl.ds(0, _BPW)])

    def _fetch(ju, jt, b):
        cu = pl.multiple_of(jnp.bitwise_and(ju, -128), 128)
        ct = pl.multiple_of(jnp.bitwise_and(jt, -128), 128)
        pltpu.async_copy(ut_hbm.at[:, pl.ds(cu, 128)], ubuf.at[b], usems[b])
        pltpu.async_copy(vt_hbm.at[:, pl.ds(ct, 128)], tbuf.at[b], tsems[b])

    def _drain(b):
        pltpu.make_async_copy(ut_hbm.at[:, pl.ds(0, 128)], ubuf.at[b], usems[b]).wait()
        pltpu.make_async_copy(vt_hbm.at[:, pl.ds(0, 128)], tbuf.at[b], tsems[b]).wait()

    iota_lo = lax.iota(jnp.int32, LANES)
    iota_hi = iota_lo + LANES

    def _select_row(b, lane, k):
        lane_v = jnp.full((LANES,), lane, jnp.int32)
        lo = plsc.load_gather(ubuf.at[b], [iota_lo, lane_v])
        hi = plsc.load_gather(ubuf.at[b], [iota_hi, lane_v])
        uoutR[k, pl.ds(0, LANES)] = lo
        uoutR[k, pl.ds(LANES, LANES)] = hi

    def _select_col(b, lane, k):
        lane_v = jnp.full((LANES,), lane, jnp.int32)
        k_v = jnp.full((LANES,), k, jnp.int32)
        lo = plsc.load_gather(tbuf.at[b], [iota_lo, lane_v])
        hi = plsc.load_gather(tbuf.at[b], [iota_hi, lane_v])
        plsc.store_scatter(toutT, [iota_lo, k_v], lo)
        plsc.store_scatter(toutT, [iota_hi, k_v], hi)

    uvec0 = uidx_v[pl.ds(0, LANES)]
    tvec0 = tidx_v[pl.ds(0, LANES)]
    for b in range(_NB):
        _fetch(uvec0[b], tvec0[b], b)

    def group(g, carry):
        uvec = uidx_v[pl.ds(g * _NB, LANES)]
        tvec = tidx_v[pl.ds(g * _NB, LANES)]
        for b in range(_NB):
            k = g * _NB + b
            _drain(b)
            _select_row(b, jnp.bitwise_and(uvec[b], 127), k)
            _select_col(b, jnp.bitwise_and(tvec[b], 127), k)

            @pl.when(k + _NB < _BPW)
            def _():
                _fetch(uvec[b + _NB], tvec[b + _NB], b)

        return carry

    lax.fori_loop(0, _NG, group, 0)
    pltpu.sync_copy(uoutR, uout_hbm.at[pl.ds(base, _BPW), :])
    pltpu.sync_copy(toutT, tout_hbm.at[:, pl.ds(base, _BPW)])


def _mm_body(a_ref, b_ref, o_ref):
    o_ref[...] = lax.dot_general(
        a_ref[...], b_ref[...],
        dimension_numbers=(((1,), (0,)), ((), ())),
        preferred_element_type=jnp.float32,
    )


_BM = 512


def _tc_matmul(u_ebd, t_ebd_t):
    return pl.pallas_call(
        _mm_body,
        grid=(B // _BM,),
        in_specs=[
            pl.BlockSpec((_BM, D), lambda i: (i, 0)),
            pl.BlockSpec((D, B), lambda i: (0, 0)),
        ],
        out_specs=pl.BlockSpec((_BM, B), lambda i: (i, 0)),
        out_shape=jax.ShapeDtypeStruct((B, B), jnp.float32),
    )(u_ebd, t_ebd_t)


def kernel(user_indices, item_seq_indices, target_item_indices, target_domain, U, V):
    uidx = user_indices.astype(jnp.int32)
    tidx = target_item_indices.reshape(B).astype(jnp.int32)
    u_ebd, t_ebd_t = _sc_gather(uidx, tidx, U.T, V.T)
    return _tc_matmul(u_ebd, t_ebd_t)
